# baseline (speedup 1.0000x reference)
import jax
import jax.numpy as jnp
import numpy as np
from jax import lax
from jax.experimental import pallas as pl
from jax.experimental.pallas import tpu as pltpu
from jax.experimental.pallas import tpu_sc as plsc

F32 = jnp.float32

NC = 2
NS = 16
NW = NC * NS
LANE = 16
WIN = 64
NBUF = 4


def _sc_mesh():
    return plsc.VectorSubcoreMesh(
        core_axis_name="c", subcore_axis_name="s",
        num_cores=NC, num_subcores=NS)


def _deg_call(er, n_hist, uni):
    nwin = er.shape[1]
    rpt = n_hist // NS

    def body(er_hbm, deg_hbm, idx_v, ones_v, zbuf_v, hist_sh, sem0, sem1):
        c = lax.axis_index("c")
        s = lax.axis_index("s")
        wid = c * NS + s

        def _z(i, _):
            zbuf_v[pl.ds(i * LANE, LANE)] = jnp.zeros((LANE,), F32)
            return 0
        lax.fori_loop(0, rpt // LANE, _z, 0)
        for i in range(WIN // LANE):
            ones_v[pl.ds(i * LANE, LANE)] = jnp.ones((LANE,), F32)
        pltpu.sync_copy(zbuf_v, hist_sh.at[pl.ds(s * rpt, rpt)])
        pltpu.sync_copy(er_hbm.at[1, pl.ds(wid * uni, uni)], idx_v)
        plsc.subcore_barrier()

        def _w(i, _):
            w = 2 * i
            c0 = pltpu.async_copy(ones_v, hist_sh.at[idx_v.at[w]],
                                  sem0, add=True)
            c1 = pltpu.async_copy(ones_v, hist_sh.at[idx_v.at[w + 1]],
                                  sem1, add=True)
            c0.wait()
            c1.wait()
            return 0
        lax.fori_loop(0, uni // 2, _w, 0)
        t_base = NW * uni
        full_t = (nwin - t_base) // 8
        rem_w = nwin - t_base - full_t * 8
        if full_t:
            @pl.when(wid < full_t)
            def _():
                pltpu.sync_copy(er_hbm.at[1, pl.ds(t_base + 8 * wid, 8)],
                                idx_v.at[pl.ds(0, 8)])
                for w in range(8):
                    pltpu.sync_copy(ones_v, hist_sh.at[idx_v.at[w]],
                                    add=True)
        if rem_w:
            @pl.when(wid == full_t)
            def _():
                pltpu.sync_copy(
                    er_hbm.at[1, pl.ds(t_base + 8 * full_t, rem_w)],
                    idx_v.at[pl.ds(0, rem_w)])
                for w in range(rem_w):
                    pltpu.sync_copy(ones_v, hist_sh.at[idx_v.at[w]],
                                    add=True)
        plsc.subcore_barrier()
        pltpu.sync_copy(hist_sh.at[pl.ds(s * rpt, rpt)],
                        deg_hbm.at[c, pl.ds(s * rpt, rpt)])

    return pl.kernel(
        body,
        out_type=jax.ShapeDtypeStruct((NC, n_hist), F32),
        mesh=_sc_mesh(),
        scratch_types=[
            pltpu.VMEM((uni, WIN), jnp.int32),
            pltpu.VMEM((WIN,), F32),
            pltpu.VMEM((rpt,), F32),
            pltpu.VMEM_SHARED((n_hist,), F32),
            pltpu.SemaphoreType.DMA,
            pltpu.SemaphoreType.DMA,
        ],
    )(er)


def _scatter_call(hs, er, n_acc, uni, ch):
    d = hs.shape[1]
    nwin = er.shape[1]
    nch = uni // ch
    rpt = n_acc // NS

    def body(hs_hbm, er_hbm, p_hbm,
             ibs, ibd, rows0, rows1, rows2, rows3, acc_sh,
             gs0, gs1, gs2, gs3, ss0, ss1, ss2, ss3, isem):
        c = lax.axis_index("c")
        s = lax.axis_index("s")
        wid = c * NS + s
        wb = wid * uni
        rows = (rows0, rows1, rows2, rows3)
        gsem = (gs0, gs1, gs2, gs3)
        ssem = (ss0, ss1, ss2, ss3)

        def _gwait(p):
            pltpu.make_async_copy(
                hs_hbm.at[pl.ds(0, WIN)], rows[p], gsem[p]).wait()

        def _swait(p):
            pltpu.make_async_copy(
                rows[p], acc_sh.at[pl.ds(0, WIN)], ssem[p]).wait()

        def _iwait(q):
            pltpu.make_async_copy(
                er_hbm.at[0, pl.ds(0, ch)], ibs.at[q], isem).wait()
            pltpu.make_async_copy(
                er_hbm.at[0, pl.ds(0, ch)], ibd.at[q], isem).wait()

        def _zr(i, _):
            def _zc(j, _):
                for r in rows:
                    r[i, pl.ds(j * LANE, LANE)] = jnp.zeros((LANE,), F32)
                return 0
            lax.fori_loop(0, d // LANE, _zc, 0)
            return 0
        lax.fori_loop(0, WIN, _zr, 0)

        def _za(m, _):
            pltpu.sync_copy(rows0, acc_sh.at[pl.ds(s * rpt + m * WIN, WIN)])
            return 0
        lax.fori_loop(0, rpt // WIN, _za, 0)

        for i in range(WIN // LANE):
            ibd[0, 0, pl.ds(i * LANE, LANE)] = (
                lax.iota(jnp.int32, LANE) + i * LANE)
        for r in range(NBUF):
            pltpu.async_copy(rows[r], acc_sh.at[ibd.at[0, 0]],
                             ssem[r], add=True)
        pltpu.async_copy(er_hbm.at[0, pl.ds(wb, ch)], ibs.at[0], isem)
        pltpu.async_copy(er_hbm.at[1, pl.ds(wb, ch)], ibd.at[0], isem)
        plsc.subcore_barrier()

        def _chunk(cc, _):
            q = cc & 1
            nxt = jnp.minimum(cc + 1, nch - 1)
            _iwait(q)
            for j in range(ch):
                p = j % NBUF
                pv = (j - 1) % NBUF
                _swait(p)
                pltpu.async_copy(
                    hs_hbm.at[ibs.at[q, j]], rows[p], gsem[p])
                if j == 0:
                    @pl.when(cc > 0)
                    def _():
                        _gwait(pv)
                        pltpu.async_copy(
                            rows[pv], acc_sh.at[ibd.at[1 - q, ch - 1]],
                            ssem[pv], add=True)
                else:
                    _gwait(pv)
                    pltpu.async_copy(
                        rows[pv], acc_sh.at[ibd.at[q, j - 1]],
                        ssem[pv], add=True)
                if j == 1:
                    pltpu.async_copy(
                        er_hbm.at[0, pl.ds(wb + nxt * ch, ch)],
                        ibs.at[1 - q], isem)
                    pltpu.async_copy(
                        er_hbm.at[1, pl.ds(wb + nxt * ch, ch)],
                        ibd.at[1 - q], isem)
            return 0
        lax.fori_loop(0, nch, _chunk, 0)

        p_last = (uni - 1) % NBUF
        q_last = (nch - 1) & 1
        _gwait(p_last)
        pltpu.async_copy(
            rows[p_last], acc_sh.at[ibd.at[q_last, ch - 1]],
            ssem[p_last], add=True)
        for r in range(NBUF):
            _swait(r)
        _iwait(1 - q_last)

        t_base = NW * uni
        full_t = (nwin - t_base) // 8
        rem_w = nwin - t_base - full_t * 8

        def _tail_chunk(start, k):
            pltpu.sync_copy(er_hbm.at[0, pl.ds(start, k)],
                            ibs.at[0, pl.ds(0, k)])
            pltpu.sync_copy(er_hbm.at[1, pl.ds(start, k)],
                            ibd.at[0, pl.ds(0, k)])
            cps = [None, None]
            scs = [None, None]
            for w in range(k):
                p = w & 1
                if w >= 2:
                    scs[p].wait()
                cps[p] = pltpu.async_copy(
                    hs_hbm.at[ibs.at[0, w]], rows[p], gsem[p])
                if w >= 1:
                    cps[1 - p].wait()
                    scs[1 - p] = pltpu.async_copy(
                        rows[1 - p], acc_sh.at[ibd.at[0, w - 1]],
                        ssem[1 - p], add=True)
            pq = (k - 1) & 1
            cps[pq].wait()
            if k >= 2:
                scs[1 - pq].wait()
            sc = pltpu.async_copy(rows[pq], acc_sh.at[ibd.at[0, k - 1]],
                                  ssem[pq], add=True)
            sc.wait()

        if full_t:
            @pl.when(wid < full_t)
            def _():
                _tail_chunk(t_base + 8 * wid, 8)
        if rem_w:
            @pl.when(wid == full_t)
            def _():
                _tail_chunk(t_base + 8 * full_t, rem_w)
        plsc.subcore_barrier()

        pltpu.sync_copy(acc_sh.at[pl.ds(s * rpt, rpt)],
                        p_hbm.at[c, pl.ds(s * rpt, rpt)])

    return pl.kernel(
        body,
        out_type=jax.ShapeDtypeStruct((NC, n_acc, d), F32),
        mesh=_sc_mesh(),
        scratch_types=[
            pltpu.VMEM((2, ch, WIN), jnp.int32),
            pltpu.VMEM((2, ch, WIN), jnp.int32),
            pltpu.VMEM((WIN, d), F32),
            pltpu.VMEM((WIN, d), F32),
            pltpu.VMEM((WIN, d), F32),
            pltpu.VMEM((WIN, d), F32),
            pltpu.VMEM_SHARED((n_acc, d), F32),
        ] + [pltpu.SemaphoreType.DMA] * 9,
    )(hs, er)


_ROWS = 2000


def _mm_body(x_ref, w_ref, h_ref):
    h_ref[...] = jnp.dot(x_ref[...], w_ref[...], preferred_element_type=F32)


def _mm_call(x, W1):
    n, d = x.shape
    h = W1.shape[1]
    return pl.pallas_call(
        _mm_body,
        grid=(n // _ROWS,),
        in_specs=[
            pl.BlockSpec((_ROWS, d), lambda i: (i, 0)),
            pl.BlockSpec((d, h), lambda i: (0, 0)),
        ],
        out_specs=pl.BlockSpec((_ROWS, h), lambda i: (i, 0)),
        out_shape=jax.ShapeDtypeStruct((n, h), F32),
    )(x, W1)


def _comb_body(h_ref, d0_ref, d1_ref, hs_ref, dinv_ref):
    dinv = lax.rsqrt(d0_ref[...] + d1_ref[...] + 1.0)
    hs_ref[...] = h_ref[...] * dinv
    dinv_ref[...] = dinv


def _comb_call(h1, d0, d1):
    n, h = h1.shape
    return pl.pallas_call(
        _comb_body,
        grid=(n // _ROWS,),
        in_specs=[
            pl.BlockSpec((_ROWS, h), lambda i: (i, 0)),
            pl.BlockSpec((_ROWS, 1), lambda i: (i, 0)),
            pl.BlockSpec((_ROWS, 1), lambda i: (i, 0)),
        ],
        out_specs=[
            pl.BlockSpec((_ROWS, h), lambda i: (i, 0)),
            pl.BlockSpec((_ROWS, 1), lambda i: (i, 0)),
        ],
        out_shape=[
            jax.ShapeDtypeStruct((n, h), F32),
            jax.ShapeDtypeStruct((n, 1), F32),
        ],
    )(h1, d0, d1)


def _mid_body(p_ref, hs_ref, dinv_ref, b_ref, w_ref, hs2_ref):
    dinv = dinv_ref[...]
    z = jnp.maximum(
        dinv * (p_ref[0] + p_ref[1] + hs_ref[...]) + b_ref[...], 0.0)
    hs2_ref[...] = jnp.dot(z, w_ref[...], preferred_element_type=F32) * dinv


def _mid_call(P, hs1, dinv, b1, W2):
    n, h = hs1.shape
    return pl.pallas_call(
        _mid_body,
        grid=(n // _ROWS,),
        in_specs=[
            pl.BlockSpec((NC, _ROWS, h), lambda i: (0, i, 0)),
            pl.BlockSpec((_ROWS, h), lambda i: (i, 0)),
            pl.BlockSpec((_ROWS, 1), lambda i: (i, 0)),
            pl.BlockSpec((1, h), lambda i: (0, 0)),
            pl.BlockSpec((h, h), lambda i: (0, 0)),
        ],
        out_specs=pl.BlockSpec((_ROWS, h), lambda i: (i, 0)),
        out_shape=jax.ShapeDtypeStruct((n, h), F32),
    )(P, hs1, dinv, b1, W2)


def _fin_body(p_ref, hs_ref, dinv_ref, b_ref, o_ref):
    dinv = dinv_ref[...]
    o_ref[...] = jnp.maximum(
        dinv * (p_ref[0] + p_ref[1] + hs_ref[...]) + b_ref[...], 0.0)


def _fin_call(P, hs2, dinv, b2):
    n, h = hs2.shape
    return pl.pallas_call(
        _fin_body,
        grid=(n // _ROWS,),
        in_specs=[
            pl.BlockSpec((NC, _ROWS, h), lambda i: (0, i, 0)),
            pl.BlockSpec((_ROWS, h), lambda i: (i, 0)),
            pl.BlockSpec((_ROWS, 1), lambda i: (i, 0)),
            pl.BlockSpec((1, h), lambda i: (0, 0)),
        ],
        out_specs=pl.BlockSpec((_ROWS, h), lambda i: (i, 0)),
        out_shape=jax.ShapeDtypeStruct((n, h), F32),
    )(P, hs2, dinv, b2)


def _round_up(a, b):
    return -(-a // b) * b


def kernel(x, edge_index, W1, b1, W2, b2):
    n, d = x.shape
    h = W1.shape[1]
    e = edge_index.shape[1]

    n_acc = _round_up(n + 64, NS * WIN)
    e_pad = _round_up(e, WIN)
    if e_pad != e:
        pad_i = np.arange(e_pad - e, dtype=np.int32)
        edge_index = jnp.concatenate(
            [edge_index,
             jnp.stack([jnp.asarray(pad_i % n),
                        jnp.asarray(n + pad_i % (n_acc - n))])], axis=1)
    nwin = e_pad // WIN
    uni = (nwin // NW) & ~7
    ch = 8

    er = edge_index.reshape(2, nwin, WIN)

    deg = _deg_call(er, n_acc, uni)
    h1 = _mm_call(x, W1)
    hs1, dinv = _comb_call(h1, deg[0].reshape(n_acc, 1),
                           deg[1].reshape(n_acc, 1))
    P1 = _scatter_call(hs1, er, n_acc, uni, ch)
    hs2 = _mid_call(P1, hs1, dinv, b1.reshape(1, h), W2)
    P2 = _scatter_call(hs2, er, n_acc, uni, ch)
    return _fin_call(P2, hs2, dinv, b2.reshape(1, h))


# speedup vs baseline: 1.1140x; 1.1140x over previous
import jax
import jax.numpy as jnp
import numpy as np
from jax import lax
from jax.experimental import pallas as pl
from jax.experimental.pallas import tpu as pltpu
from jax.experimental.pallas import tpu_sc as plsc

F32 = jnp.float32

NC = 2
NS = 16
NW = NC * NS
LANE = 16
WIN = 128


def _sc_mesh():
    return plsc.VectorSubcoreMesh(
        core_axis_name="c", subcore_axis_name="s",
        num_cores=NC, num_subcores=NS)


def _deg_call(er, n_hist, uni):
    nwin = er.shape[1]
    rpt = n_hist // NS

    def body(er_hbm, deg_hbm, idx_v, ones_v, zbuf_v, hist_sh, sem0, sem1):
        c = lax.axis_index("c")
        s = lax.axis_index("s")
        wid = c * NS + s

        def _z(i, _):
            zbuf_v[pl.ds(i * LANE, LANE)] = jnp.zeros((LANE,), F32)
            return 0
        lax.fori_loop(0, rpt // LANE, _z, 0)
        for i in range(WIN // LANE):
            ones_v[pl.ds(i * LANE, LANE)] = jnp.ones((LANE,), F32)
        pltpu.sync_copy(zbuf_v, hist_sh.at[pl.ds(s * rpt, rpt)])
        pltpu.sync_copy(er_hbm.at[1, pl.ds(wid * uni, uni)], idx_v)
        plsc.subcore_barrier()

        def _w(i, _):
            w = 2 * i
            c0 = pltpu.async_copy(ones_v, hist_sh.at[idx_v.at[w]],
                                  sem0, add=True)
            c1 = pltpu.async_copy(ones_v, hist_sh.at[idx_v.at[w + 1]],
                                  sem1, add=True)
            c0.wait()
            c1.wait()
            return 0
        lax.fori_loop(0, uni // 2, _w, 0)
        t_base = NW * uni
        full_t = (nwin - t_base) // 8
        rem_w = nwin - t_base - full_t * 8
        if full_t:
            @pl.when(wid < full_t)
            def _():
                pltpu.sync_copy(er_hbm.at[1, pl.ds(t_base + 8 * wid, 8)],
                                idx_v.at[pl.ds(0, 8)])
                for w in range(8):
                    pltpu.sync_copy(ones_v, hist_sh.at[idx_v.at[w]],
                                    add=True)
        if rem_w:
            @pl.when(wid == full_t)
            def _():
                pltpu.sync_copy(
                    er_hbm.at[1, pl.ds(t_base + 8 * full_t, rem_w)],
                    idx_v.at[pl.ds(0, rem_w)])
                for w in range(rem_w):
                    pltpu.sync_copy(ones_v, hist_sh.at[idx_v.at[w]],
                                    add=True)
        plsc.subcore_barrier()
        pltpu.sync_copy(hist_sh.at[pl.ds(s * rpt, rpt)],
                        deg_hbm.at[c, pl.ds(s * rpt, rpt)])

    return pl.kernel(
        body,
        out_type=jax.ShapeDtypeStruct((NC, n_hist), F32),
        mesh=_sc_mesh(),
        scratch_types=[
            pltpu.VMEM((uni, WIN), jnp.int32),
            pltpu.VMEM((WIN,), F32),
            pltpu.VMEM((rpt,), F32),
            pltpu.VMEM_SHARED((n_hist,), F32),
            pltpu.SemaphoreType.DMA,
            pltpu.SemaphoreType.DMA,
        ],
    )(er)


def _scatter_call(hs, er, n_acc, uni, ch):
    d = hs.shape[1]
    nwin = er.shape[1]
    nch = uni // ch
    rpt = n_acc // NS

    def body(hs_hbm, er_hbm, p_hbm,
             ibs, ibd, rows0, rows1, acc_sh, gs0, gs1, ss0, ss1, isem):
        c = lax.axis_index("c")
        s = lax.axis_index("s")
        wid = c * NS + s
        wb = wid * uni
        rows = (rows0, rows1)
        gsem = (gs0, gs1)
        ssem = (ss0, ss1)

        def _gwait(p):
            pltpu.make_async_copy(
                hs_hbm.at[pl.ds(0, WIN)], rows[p], gsem[p]).wait()

        def _swait(p):
            pltpu.make_async_copy(
                rows[p], acc_sh.at[pl.ds(0, WIN)], ssem[p]).wait()

        def _iwait(q):
            pltpu.make_async_copy(
                er_hbm.at[0, pl.ds(0, ch)], ibs.at[q], isem).wait()
            pltpu.make_async_copy(
                er_hbm.at[0, pl.ds(0, ch)], ibd.at[q], isem).wait()

        def _zr(i, _):
            def _zc(j, _):
                rows0[i, pl.ds(j * LANE, LANE)] = jnp.zeros((LANE,), F32)
                rows1[i, pl.ds(j * LANE, LANE)] = jnp.zeros((LANE,), F32)
                return 0
            lax.fori_loop(0, d // LANE, _zc, 0)
            return 0
        lax.fori_loop(0, WIN, _zr, 0)

        def _za(m, _):
            pltpu.sync_copy(rows0, acc_sh.at[pl.ds(s * rpt + m * WIN, WIN)])
            return 0
        lax.fori_loop(0, rpt // WIN, _za, 0)

        for i in range(WIN // LANE):
            ibd[0, 0, pl.ds(i * LANE, LANE)] = (
                lax.iota(jnp.int32, LANE) + i * LANE)
        pltpu.async_copy(rows0, acc_sh.at[ibd.at[0, 0]], ss0, add=True)
        pltpu.async_copy(rows1, acc_sh.at[ibd.at[0, 0]], ss1, add=True)
        pltpu.async_copy(er_hbm.at[0, pl.ds(wb, ch)], ibs.at[0], isem)
        pltpu.async_copy(er_hbm.at[1, pl.ds(wb, ch)], ibd.at[0], isem)
        plsc.subcore_barrier()

        def _chunk(cc, _):
            q = cc & 1
            nxt = jnp.minimum(cc + 1, nch - 1)
            _iwait(q)
            for j in range(ch):
                p = j & 1
                _swait(p)
                pltpu.async_copy(
                    hs_hbm.at[ibs.at[q, j]], rows[p], gsem[p])
                if j == 0:
                    @pl.when(cc > 0)
                    def _():
                        _gwait(1 - p)
                        pltpu.async_copy(
                            rows[1 - p], acc_sh.at[ibd.at[1 - q, ch - 1]],
                            ssem[1 - p], add=True)
                else:
                    _gwait(1 - p)
                    pltpu.async_copy(
                        rows[1 - p], acc_sh.at[ibd.at[q, j - 1]],
                        ssem[1 - p], add=True)
                if j == 1:
                    pltpu.async_copy(
                        er_hbm.at[0, pl.ds(wb + nxt * ch, ch)],
                        ibs.at[1 - q], isem)
                    pltpu.async_copy(
                        er_hbm.at[1, pl.ds(wb + nxt * ch, ch)],
                        ibd.at[1 - q], isem)
            return 0
        lax.fori_loop(0, nch, _chunk, 0)

        p_last = (uni - 1) & 1
        q_last = (nch - 1) & 1
        _gwait(p_last)
        fin = pltpu.async_copy(
            rows[p_last], acc_sh.at[ibd.at[q_last, ch - 1]],
            ssem[p_last], add=True)
        _swait(1 - p_last)
        fin.wait()
        _iwait(1 - q_last)

        t_base = NW * uni
        full_t = (nwin - t_base) // 8
        rem_w = nwin - t_base - full_t * 8

        def _tail_chunk(start, k):
            pltpu.sync_copy(er_hbm.at[0, pl.ds(start, k)],
                            ibs.at[0, pl.ds(0, k)])
            pltpu.sync_copy(er_hbm.at[1, pl.ds(start, k)],
                            ibd.at[0, pl.ds(0, k)])
            cps = [None, None]
            scs = [None, None]
            for w in range(k):
                p = w & 1
                if w >= 2:
                    scs[p].wait()
                cps[p] = pltpu.async_copy(
                    hs_hbm.at[ibs.at[0, w]], rows[p], gsem[p])
                if w >= 1:
                    cps[1 - p].wait()
                    scs[1 - p] = pltpu.async_copy(
                        rows[1 - p], acc_sh.at[ibd.at[0, w - 1]],
                        ssem[1 - p], add=True)
            pq = (k - 1) & 1
            cps[pq].wait()
            if k >= 2:
                scs[1 - pq].wait()
            sc = pltpu.async_copy(rows[pq], acc_sh.at[ibd.at[0, k - 1]],
                                  ssem[pq], add=True)
            sc.wait()

        if full_t:
            @pl.when(wid < full_t)
            def _():
                _tail_chunk(t_base + 8 * wid, 8)
        if rem_w:
            @pl.when(wid == full_t)
            def _():
                _tail_chunk(t_base + 8 * full_t, rem_w)
        plsc.subcore_barrier()

        pltpu.sync_copy(acc_sh.at[pl.ds(s * rpt, rpt)],
                        p_hbm.at[c, pl.ds(s * rpt, rpt)])

    return pl.kernel(
        body,
        out_type=jax.ShapeDtypeStruct((NC, n_acc, d), F32),
        mesh=_sc_mesh(),
        scratch_types=[
            pltpu.VMEM((2, ch, WIN), jnp.int32),
            pltpu.VMEM((2, ch, WIN), jnp.int32),
            pltpu.VMEM((WIN, d), F32),
            pltpu.VMEM((WIN, d), F32),
            pltpu.VMEM_SHARED((n_acc, d), F32),
            pltpu.SemaphoreType.DMA,
            pltpu.SemaphoreType.DMA,
            pltpu.SemaphoreType.DMA,
            pltpu.SemaphoreType.DMA,
            pltpu.SemaphoreType.DMA,
        ],
    )(hs, er)


_ROWS = 2000


def _mm_body(x_ref, w_ref, h_ref):
    h_ref[...] = jnp.dot(x_ref[...], w_ref[...], preferred_element_type=F32)


def _mm_call(x, W1):
    n, d = x.shape
    h = W1.shape[1]
    return pl.pallas_call(
        _mm_body,
        grid=(n // _ROWS,),
        in_specs=[
            pl.BlockSpec((_ROWS, d), lambda i: (i, 0)),
            pl.BlockSpec((d, h), lambda i: (0, 0)),
        ],
        out_specs=pl.BlockSpec((_ROWS, h), lambda i: (i, 0)),
        out_shape=jax.ShapeDtypeStruct((n, h), F32),
    )(x, W1)


def _comb_body(h_ref, d0_ref, d1_ref, hs_ref, dinv_ref):
    dinv = lax.rsqrt(d0_ref[...] + d1_ref[...] + 1.0)
    hs_ref[...] = h_ref[...] * dinv
    dinv_ref[...] = dinv


def _comb_call(h1, d0, d1):
    n, h = h1.shape
    return pl.pallas_call(
        _comb_body,
        grid=(n // _ROWS,),
        in_specs=[
            pl.BlockSpec((_ROWS, h), lambda i: (i, 0)),
            pl.BlockSpec((_ROWS, 1), lambda i: (i, 0)),
            pl.BlockSpec((_ROWS, 1), lambda i: (i, 0)),
        ],
        out_specs=[
            pl.BlockSpec((_ROWS, h), lambda i: (i, 0)),
            pl.BlockSpec((_ROWS, 1), lambda i: (i, 0)),
        ],
        out_shape=[
            jax.ShapeDtypeStruct((n, h), F32),
            jax.ShapeDtypeStruct((n, 1), F32),
        ],
    )(h1, d0, d1)


def _mid_body(p_ref, hs_ref, dinv_ref, b_ref, w_ref, hs2_ref):
    dinv = dinv_ref[...]
    z = jnp.maximum(
        dinv * (p_ref[0] + p_ref[1] + hs_ref[...]) + b_ref[...], 0.0)
    hs2_ref[...] = jnp.dot(z, w_ref[...], preferred_element_type=F32) * dinv


def _mid_call(P, hs1, dinv, b1, W2):
    n, h = hs1.shape
    return pl.pallas_call(
        _mid_body,
        grid=(n // _ROWS,),
        in_specs=[
            pl.BlockSpec((NC, _ROWS, h), lambda i: (0, i, 0)),
            pl.BlockSpec((_ROWS, h), lambda i: (i, 0)),
            pl.BlockSpec((_ROWS, 1), lambda i: (i, 0)),
            pl.BlockSpec((1, h), lambda i: (0, 0)),
            pl.BlockSpec((h, h), lambda i: (0, 0)),
        ],
        out_specs=pl.BlockSpec((_ROWS, h), lambda i: (i, 0)),
        out_shape=jax.ShapeDtypeStruct((n, h), F32),
    )(P, hs1, dinv, b1, W2)


def _fin_body(p_ref, hs_ref, dinv_ref, b_ref, o_ref):
    dinv = dinv_ref[...]
    o_ref[...] = jnp.maximum(
        dinv * (p_ref[0] + p_ref[1] + hs_ref[...]) + b_ref[...], 0.0)


def _fin_call(P, hs2, dinv, b2):
    n, h = hs2.shape
    return pl.pallas_call(
        _fin_body,
        grid=(n // _ROWS,),
        in_specs=[
            pl.BlockSpec((NC, _ROWS, h), lambda i: (0, i, 0)),
            pl.BlockSpec((_ROWS, h), lambda i: (i, 0)),
            pl.BlockSpec((_ROWS, 1), lambda i: (i, 0)),
            pl.BlockSpec((1, h), lambda i: (0, 0)),
        ],
        out_specs=pl.BlockSpec((_ROWS, h), lambda i: (i, 0)),
        out_shape=jax.ShapeDtypeStruct((n, h), F32),
    )(P, hs2, dinv, b2)


def _round_up(a, b):
    return -(-a // b) * b


def kernel(x, edge_index, W1, b1, W2, b2):
    n, d = x.shape
    h = W1.shape[1]
    e = edge_index.shape[1]

    n_acc = _round_up(n + 64, NS * WIN)
    e_pad = _round_up(e, WIN)
    if e_pad != e:
        pad_i = np.arange(e_pad - e, dtype=np.int32)
        edge_index = jnp.concatenate(
            [edge_index,
             jnp.stack([jnp.asarray(pad_i % n),
                        jnp.asarray(n + pad_i % (n_acc - n))])], axis=1)
    nwin = e_pad // WIN
    uni = (nwin // NW) & ~7
    ch = 8

    er = edge_index.reshape(2, nwin, WIN)

    deg = _deg_call(er, n_acc, uni)
    h1 = _mm_call(x, W1)
    hs1, dinv = _comb_call(h1, deg[0].reshape(n_acc, 1),
                           deg[1].reshape(n_acc, 1))
    P1 = _scatter_call(hs1, er, n_acc, uni, ch)
    hs2 = _mid_call(P1, hs1, dinv, b1.reshape(1, h), W2)
    P2 = _scatter_call(hs2, er, n_acc, uni, ch)
    return _fin_call(P2, hs2, dinv, b2.reshape(1, h))
